# baseline (speedup 1.0000x reference)
import jax
import jax.numpy as jnp
from jax import lax
from jax.experimental import pallas as pl

N = 16384
D = 32
K = 8192
EPS = 1e-12
BN = 512
NB = N // BN


def _main_body(x_ref, c_ref, rv_ref, xsq_ref, c2_ref,
               quant_ref, idx_ref, counts_ref):
    x = x_ref[...]
    c = c_ref[...]
    rv = rv_ref[...]
    xsq = xsq_ref[...]
    c2 = c2_ref[...]
    s = lax.dot_general(x, c, (((1,), (1,)), ((), ())),
                        preferred_element_type=jnp.float32)
    dist = (xsq - 2.0 * s) + c2
    m = jnp.min(dist, axis=1, keepdims=True)
    kiota = lax.broadcasted_iota(jnp.int32, dist.shape, 1)
    cand = jnp.where(dist == m, kiota, K)
    idx = jnp.min(cand, axis=1, keepdims=True)
    resid = jnp.sqrt(jnp.maximum(m, 0.0))
    rvn = jnp.sqrt(jnp.sum(rv * rv, axis=1, keepdims=True))
    quant_ref[...] = x + (resid / (rvn + EPS)) * rv
    idx_ref[...] = idx

    @pl.when(pl.program_id(0) == 0)
    def _():
        counts_ref[...] = jnp.zeros_like(counts_ref)

    onehot = (kiota == idx).astype(jnp.int32)
    counts_ref[...] += jnp.sum(onehot, axis=0, keepdims=True)


def _perplexity_body(counts_ref, perp_ref):
    p = counts_ref[...].astype(jnp.float32) / float(N)
    ent = -jnp.sum(p * jnp.log(p + EPS), keepdims=True)
    perp_ref[...] = jnp.exp(ent)


def kernel(input_data, codebooks):
    xsq = jnp.sum(input_data ** 2, axis=1, keepdims=True)
    c2 = jnp.sum(codebooks.T ** 2, axis=0, keepdims=True)
    rv = jax.random.normal(jax.random.key(1234), input_data.shape,
                           dtype=jnp.float32)
    quant, idx, counts = pl.pallas_call(
        _main_body,
        grid=(NB,),
        in_specs=[
            pl.BlockSpec((BN, D), lambda i: (i, 0)),
            pl.BlockSpec((K, D), lambda i: (0, 0)),
            pl.BlockSpec((BN, D), lambda i: (i, 0)),
            pl.BlockSpec((BN, 1), lambda i: (i, 0)),
            pl.BlockSpec((1, K), lambda i: (0, 0)),
        ],
        out_specs=[
            pl.BlockSpec((BN, D), lambda i: (i, 0)),
            pl.BlockSpec((BN, 1), lambda i: (i, 0)),
            pl.BlockSpec((1, K), lambda i: (0, 0)),
        ],
        out_shape=[
            jax.ShapeDtypeStruct((N, D), jnp.float32),
            jax.ShapeDtypeStruct((N, 1), jnp.int32),
            jax.ShapeDtypeStruct((1, K), jnp.int32),
        ],
    )(input_data, codebooks, rv, xsq, c2)

    perp = pl.pallas_call(
        _perplexity_body,
        in_specs=[pl.BlockSpec((1, K), lambda: (0, 0))],
        out_specs=pl.BlockSpec((1, 1), lambda: (0, 0)),
        out_shape=jax.ShapeDtypeStruct((1, 1), jnp.float32),
    )(counts)

    del idx
    return (quant, perp.reshape(()), counts.reshape(K))
